# Initial kernel scaffold; baseline (speedup 1.0000x reference)
import numpy as np

import jax
import jax.numpy as jnp
from jax.experimental import pallas as pl
from jax.experimental.pallas import tpu as pltpu

_L = 2048
_NF = 1024
_K = 5
_DT = 512


def _build_table() -> np.ndarray:
    t = np.arange(_L, dtype=np.float64)
    f = np.arange(1, _NF + 1, dtype=np.float64)
    ang = (2.0 * np.pi / _L) * np.outer(t, f)
    return np.concatenate([np.cos(ang), np.sin(ang)], axis=1).astype(np.float32)


_Q_TABLE = _build_table()


def _freq_attn_kernel(q_ref, z_ref, o_ref):
    Q = q_ref[...]
    Z = z_ref[0]
    ReIm = jax.lax.dot_general(
        Q, Z, (((0,), (0,)), ((), ())),
        preferred_element_type=jnp.float32,
        precision=jax.lax.Precision.HIGHEST)
    Re = ReIm[:_NF]
    Im = ReIm[_NF:]
    amp = jnp.sqrt(Re * Re + Im * Im)
    fidx = jax.lax.broadcasted_iota(jnp.int32, amp.shape, 0)

    def body(_, w):
        m = jnp.max(w, axis=0, keepdims=True)
        idx = jnp.min(jnp.where(w == m, fidx, _NF), axis=0, keepdims=True)
        return jnp.where(fidx == idx, -1.0, w)

    work = jax.lax.fori_loop(0, _K, body, amp)
    sel = work < 0.0
    spec = jnp.where(jnp.concatenate([sel, sel], axis=0), ReIm, 0.0)
    S = jax.lax.dot_general(
        Q, spec, (((1,), (0,)), ((), ())),
        preferred_element_type=jnp.float32,
        precision=jax.lax.Precision.HIGHEST)
    o_ref[0] = S


def kernel(Z):
    Bs, Ls, Ds = Z.shape
    q = jnp.asarray(_Q_TABLE)
    grid = (Bs, Ds // _DT)
    return pl.pallas_call(
        _freq_attn_kernel,
        grid=grid,
        in_specs=[
            pl.BlockSpec((_L, 2 * _NF), lambda b, j: (0, 0)),
            pl.BlockSpec((1, _L, _DT), lambda b, j: (b, 0, j)),
        ],
        out_specs=pl.BlockSpec((1, _L, _DT), lambda b, j: (b, 0, j)),
        out_shape=jax.ShapeDtypeStruct((Bs, Ls, Ds), jnp.float32),
        compiler_params=pltpu.CompilerParams(
            dimension_semantics=("parallel", "parallel")),
    )(q, Z)


# speedup vs baseline: 1.6696x; 1.6696x over previous
import numpy as np

import jax
import jax.numpy as jnp
from jax.experimental import pallas as pl
from jax.experimental.pallas import tpu as pltpu

_L = 2048
_NF = 1024
_K = 5
_DT = 256


def _build_table() -> np.ndarray:
    t = np.arange(_L, dtype=np.float64)
    f = np.arange(1, _NF + 1, dtype=np.float64)
    ang = (2.0 * np.pi / _L) * np.outer(t, f)
    return np.concatenate([np.cos(ang), np.sin(ang)], axis=1).astype(np.float32)


_Q_TABLE = _build_table()


def _freq_attn_kernel(q_ref, z_ref, o_ref):
    Q = q_ref[...]
    Z = z_ref[0]
    ReIm = jax.lax.dot_general(
        Q, Z, (((0,), (0,)), ((), ())),
        preferred_element_type=jnp.float32,
        precision=jax.lax.Precision.HIGHEST)
    Re = ReIm[:_NF]
    Im = ReIm[_NF:]
    amp = jnp.sqrt(Re * Re + Im * Im)
    fidx = jax.lax.broadcasted_iota(jnp.int32, amp.shape, 0)

    def body(_, w):
        m = jnp.max(w, axis=0, keepdims=True)
        idx = jnp.min(jnp.where(w == m, fidx, _NF), axis=0, keepdims=True)
        return jnp.where(fidx == idx, -1.0, w)

    work = jax.lax.fori_loop(0, _K, body, amp)
    sel = work < 0.0
    spec = jnp.where(jnp.concatenate([sel, sel], axis=0), ReIm, 0.0)
    S = jax.lax.dot_general(
        Q, spec, (((1,), (0,)), ((), ())),
        preferred_element_type=jnp.float32,
        precision=jax.lax.Precision.HIGHEST)
    o_ref[0] = S


def kernel(Z):
    Bs, Ls, Ds = Z.shape
    q = jnp.asarray(_Q_TABLE)
    grid = (Bs, Ds // _DT)
    return pl.pallas_call(
        _freq_attn_kernel,
        grid=grid,
        in_specs=[
            pl.BlockSpec((_L, 2 * _NF), lambda b, j: (0, 0)),
            pl.BlockSpec((1, _L, _DT), lambda b, j: (b, 0, j)),
        ],
        out_specs=pl.BlockSpec((1, _L, _DT), lambda b, j: (b, 0, j)),
        out_shape=jax.ShapeDtypeStruct((Bs, Ls, Ds), jnp.float32),
        compiler_params=pltpu.CompilerParams(
            dimension_semantics=("parallel", "parallel")),
    )(q, Z)


# speedup vs baseline: 5.3658x; 3.2138x over previous
import numpy as np

import jax
import jax.numpy as jnp
from jax.experimental import pallas as pl
from jax.experimental.pallas import tpu as pltpu

_L = 2048
_NF = 1024
_K = 5
_TC = 256
_FC = 256


def _build_table() -> np.ndarray:
    t = np.arange(_L, dtype=np.float64)
    f = np.arange(1, _NF + 1, dtype=np.float64)
    ang = (2.0 * np.pi / _L) * np.outer(t, f)
    return np.concatenate([np.cos(ang), np.sin(ang)], axis=1).astype(np.float32)


_Q_TABLE = _build_table()


def _dft_select_kernel(q_ref, z_ref, spec_ref, acc_ref):
    tc = pl.program_id(1)
    ntc = pl.num_programs(1)

    @pl.when(tc == 0)
    def _():
        acc_ref[...] = jnp.zeros_like(acc_ref)

    acc_ref[...] += jax.lax.dot_general(
        q_ref[...], z_ref[0], (((0,), (0,)), ((), ())),
        preferred_element_type=jnp.float32,
        precision=jax.lax.Precision.HIGHEST)

    @pl.when(tc == ntc - 1)
    def _():
        ReIm = acc_ref[...]
        Re = ReIm[:_NF]
        Im = ReIm[_NF:]
        amp = jnp.sqrt(Re * Re + Im * Im)
        fidx = jax.lax.broadcasted_iota(jnp.int32, amp.shape, 0)

        def body(_, w):
            m = jnp.max(w, axis=0, keepdims=True)
            idx = jnp.min(jnp.where(w == m, fidx, _NF), axis=0, keepdims=True)
            return jnp.where(fidx == idx, -1.0, w)

        work = jax.lax.fori_loop(0, _K, body, amp)
        sel = work < 0.0
        spec = jnp.where(jnp.concatenate([sel, sel], axis=0), ReIm, 0.0)
        spec_ref[0] = spec.astype(jnp.bfloat16)


def _recon_kernel(qb_ref, spec_ref, o_ref):
    fc = pl.program_id(1)

    @pl.when(fc == 0)
    def _():
        o_ref[...] = jnp.zeros_like(o_ref)

    o_ref[0] += jax.lax.dot_general(
        qb_ref[...], spec_ref[0], (((1,), (0,)), ((), ())),
        preferred_element_type=jnp.float32)


def kernel(Z):
    Bs, Ls, Ds = Z.shape
    q = jnp.asarray(_Q_TABLE)
    qb = jnp.asarray(_Q_TABLE.astype(jnp.bfloat16))

    spec = pl.pallas_call(
        _dft_select_kernel,
        grid=(Bs, Ls // _TC),
        in_specs=[
            pl.BlockSpec((_TC, 2 * _NF), lambda b, tc: (tc, 0)),
            pl.BlockSpec((1, _TC, Ds), lambda b, tc: (b, tc, 0)),
        ],
        out_specs=pl.BlockSpec((1, 2 * _NF, Ds), lambda b, tc: (b, 0, 0)),
        out_shape=jax.ShapeDtypeStruct((Bs, 2 * _NF, Ds), jnp.bfloat16),
        scratch_shapes=[pltpu.VMEM((2 * _NF, Ds), jnp.float32)],
        compiler_params=pltpu.CompilerParams(
            dimension_semantics=("parallel", "arbitrary")),
    )(q, Z)

    return pl.pallas_call(
        _recon_kernel,
        grid=(Bs, 2 * _NF // _FC),
        in_specs=[
            pl.BlockSpec((_L, _FC), lambda b, fc: (0, fc)),
            pl.BlockSpec((1, _FC, Ds), lambda b, fc: (b, fc, 0)),
        ],
        out_specs=pl.BlockSpec((1, _L, Ds), lambda b, fc: (b, 0, 0)),
        out_shape=jax.ShapeDtypeStruct((Bs, Ls, Ds), jnp.float32),
        compiler_params=pltpu.CompilerParams(
            dimension_semantics=("parallel", "arbitrary")),
    )(qb, spec)


# speedup vs baseline: 9.0585x; 1.6882x over previous
import numpy as np

import jax
import jax.numpy as jnp
from jax.experimental import pallas as pl
from jax.experimental.pallas import tpu as pltpu

_L = 2048
_NF = 1024
_K = 5
_NTC = 2
_FC = 256


def _build_tables():
    n1 = np.arange(1024, dtype=np.float64)[:, None]
    j1 = np.arange(512, dtype=np.float64)[None, :]
    ang_o = (2.0 * np.pi / 2048.0) * (2.0 * j1 + 1.0) * n1
    t_odd = np.concatenate([np.cos(ang_o), np.sin(ang_o)], axis=1)

    n2 = np.arange(512, dtype=np.float64)[:, None]
    j2 = np.arange(256, dtype=np.float64)[None, :]
    ang_e2 = (2.0 * np.pi / 1024.0) * (2.0 * j2 + 1.0) * n2
    t_e2 = np.concatenate([np.cos(ang_e2), np.sin(ang_e2)], axis=1)
    ang_e4 = (2.0 * np.pi / 512.0) * (j2 + 1.0) * n2
    t_e4 = np.concatenate([np.cos(ang_e4), np.sin(ang_e4)], axis=1)

    perm_f = np.concatenate([2 * np.arange(512) + 1,
                             4 * np.arange(256) + 2,
                             4 * np.arange(256) + 4]).astype(np.float64)
    t = np.arange(_L, dtype=np.float64)[:, None]
    ang_p = (2.0 * np.pi / _L) * perm_f[None, :] * t
    q_rec = np.concatenate([np.cos(ang_p), np.sin(ang_p)], axis=1)
    return (t_odd.astype(np.float32).reshape(2, 512, 1024),
            t_e2.astype(np.float32),
            t_e4.astype(np.float32),
            q_rec.astype(np.float32))


_T_ODD, _T_E2, _T_E4, _Q_REC = _build_tables()


def _hdot(a, b):
    return jax.lax.dot_general(
        a, b, (((0,), (0,)), ((), ())),
        preferred_element_type=jnp.float32,
        precision=jax.lax.Precision.HIGHEST)


def _dft_select_kernel(to_ref, te2_ref, te4_ref, z_ref, spec_ref,
                       odd_ref, e2_ref, e4_ref):
    tc = pl.program_id(1)
    ntc = pl.num_programs(1)

    @pl.when(tc == 0)
    def _():
        odd_ref[...] = jnp.zeros_like(odd_ref)
        e2_ref[...] = jnp.zeros_like(e2_ref)
        e4_ref[...] = jnp.zeros_like(e4_ref)

    zb = z_ref[0]
    q0, q1, q2, q3 = zb[0], zb[1], zb[2], zb[3]
    c0 = q0 - q2
    c1 = q1 - q3
    a0 = q0 + q2
    a1 = q1 + q3
    ap = a0 + a1
    cp = a0 - a1

    odd_ref[...] += _hdot(to_ref[0], c0) + _hdot(to_ref[1], c1)
    e2_ref[...] += _hdot(te2_ref[...], cp)
    e4_ref[...] += _hdot(te4_ref[...], ap)

    @pl.when(tc == ntc - 1)
    def _():
        odd = odd_ref[...]
        e2 = e2_ref[...]
        e4 = e4_ref[...]
        Re = jnp.concatenate([odd[:512], e2[:256], e4[:256]], axis=0)
        Im = jnp.concatenate([odd[512:], e2[256:], e4[256:]], axis=0)
        amp = jnp.sqrt(Re * Re + Im * Im)
        r = jax.lax.broadcasted_iota(jnp.int32, amp.shape, 0)
        fidx = jnp.where(r < 512, 2 * r + 1,
                         jnp.where(r < 768, 4 * r - 2046, 4 * r - 3068))

        def body(_, w):
            m = jnp.max(w, axis=0, keepdims=True)
            idx = jnp.min(jnp.where(w == m, fidx, 2 * _NF), axis=0,
                          keepdims=True)
            return jnp.where(fidx == idx, -1.0, w)

        work = jax.lax.fori_loop(0, _K, body, amp)
        sel = work < 0.0
        spec = jnp.where(jnp.concatenate([sel, sel], axis=0),
                         jnp.concatenate([Re, Im], axis=0), 0.0)
        spec_ref[0] = spec.astype(jnp.bfloat16)


def _recon_kernel(qb_ref, spec_ref, o_ref):
    fc = pl.program_id(1)

    @pl.when(fc == 0)
    def _():
        o_ref[...] = jnp.zeros_like(o_ref)

    o_ref[0] += jax.lax.dot_general(
        qb_ref[...], spec_ref[0], (((1,), (0,)), ((), ())),
        preferred_element_type=jnp.float32)


def kernel(Z):
    Bs, Ls, Ds = Z.shape
    to = jnp.asarray(_T_ODD)
    te2 = jnp.asarray(_T_E2)
    te4 = jnp.asarray(_T_E4)
    qb = jnp.asarray(_Q_REC.astype(jnp.bfloat16))
    zp = Z.reshape(Bs, 4, Ls // 4, Ds)
    tch = Ls // 4 // _NTC

    spec = pl.pallas_call(
        _dft_select_kernel,
        grid=(Bs, _NTC),
        in_specs=[
            pl.BlockSpec((2, tch, 1024), lambda b, tc: (0, tc, 0)),
            pl.BlockSpec((tch, 512), lambda b, tc: (tc, 0)),
            pl.BlockSpec((tch, 512), lambda b, tc: (tc, 0)),
            pl.BlockSpec((1, 4, tch, Ds), lambda b, tc: (b, 0, tc, 0)),
        ],
        out_specs=pl.BlockSpec((1, 2 * _NF, Ds), lambda b, tc: (b, 0, 0)),
        out_shape=jax.ShapeDtypeStruct((Bs, 2 * _NF, Ds), jnp.bfloat16),
        scratch_shapes=[pltpu.VMEM((_NF, Ds), jnp.float32),
                        pltpu.VMEM((512, Ds), jnp.float32),
                        pltpu.VMEM((512, Ds), jnp.float32)],
        compiler_params=pltpu.CompilerParams(
            dimension_semantics=("parallel", "arbitrary")),
    )(to, te2, te4, zp)

    return pl.pallas_call(
        _recon_kernel,
        grid=(Bs, 2 * _NF // _FC),
        in_specs=[
            pl.BlockSpec((_L, _FC), lambda b, fc: (0, fc)),
            pl.BlockSpec((1, _FC, Ds), lambda b, fc: (b, fc, 0)),
        ],
        out_specs=pl.BlockSpec((1, _L, Ds), lambda b, fc: (b, 0, 0)),
        out_shape=jax.ShapeDtypeStruct((Bs, Ls, Ds), jnp.float32),
        compiler_params=pltpu.CompilerParams(
            dimension_semantics=("parallel", "arbitrary")),
    )(qb, spec)


# speedup vs baseline: 10.6857x; 1.1796x over previous
import numpy as np

import jax
import jax.numpy as jnp
from jax.experimental import pallas as pl
from jax.experimental.pallas import tpu as pltpu

_L = 2048
_NF = 1024
_K = 5
_NTC = 2
_FC = 256


def _build_tables():
    n1 = np.arange(1024, dtype=np.float64)[:, None]
    j1 = np.arange(512, dtype=np.float64)[None, :]
    ang_o = (2.0 * np.pi / 2048.0) * (2.0 * j1 + 1.0) * n1
    t_odd = np.concatenate([np.cos(ang_o), np.sin(ang_o)], axis=1)

    n2 = np.arange(512, dtype=np.float64)[:, None]
    j2 = np.arange(256, dtype=np.float64)[None, :]
    ang_e2 = (2.0 * np.pi / 1024.0) * (2.0 * j2 + 1.0) * n2
    t_e2 = np.concatenate([np.cos(ang_e2), np.sin(ang_e2)], axis=1)
    ang_e4 = (2.0 * np.pi / 512.0) * (j2 + 1.0) * n2
    t_e4 = np.concatenate([np.cos(ang_e4), np.sin(ang_e4)], axis=1)

    return (t_odd.astype(np.float32).reshape(2, 512, 1024),
            t_e2.astype(np.float32),
            t_e4.astype(np.float32))


_T_ODD, _T_E2, _T_E4 = _build_tables()


def _hdot(a, b):
    return jax.lax.dot_general(
        a, b, (((0,), (0,)), ((), ())),
        preferred_element_type=jnp.float32,
        precision=jax.lax.Precision.HIGHEST)


def _dft_select_kernel(to_ref, te2_ref, te4_ref, z_ref, spec_ref,
                       odd_ref, e2_ref, e4_ref):
    tc = pl.program_id(1)
    ntc = pl.num_programs(1)

    @pl.when(tc == 0)
    def _():
        odd_ref[...] = jnp.zeros_like(odd_ref)
        e2_ref[...] = jnp.zeros_like(e2_ref)
        e4_ref[...] = jnp.zeros_like(e4_ref)

    zb = z_ref[0]
    q0, q1, q2, q3 = zb[0], zb[1], zb[2], zb[3]
    c0 = q0 - q2
    c1 = q1 - q3
    a0 = q0 + q2
    a1 = q1 + q3
    ap = a0 + a1
    cp = a0 - a1

    odd_ref[...] += _hdot(to_ref[0], c0) + _hdot(to_ref[1], c1)
    e2_ref[...] += _hdot(te2_ref[...], cp)
    e4_ref[...] += _hdot(te4_ref[...], ap)

    @pl.when(tc == ntc - 1)
    def _():
        odd = odd_ref[...]
        e2 = e2_ref[...]
        e4 = e4_ref[...]
        Re = jnp.concatenate([odd[:512], e2[:256], e4[:256]], axis=0)
        Im = jnp.concatenate([odd[512:], e2[256:], e4[256:]], axis=0)
        amp = jnp.sqrt(Re * Re + Im * Im)
        r = jax.lax.broadcasted_iota(jnp.int32, amp.shape, 0)
        fidx = jnp.where(r < 512, 2 * r + 1,
                         jnp.where(r < 768, 4 * r - 2046, 4 * r - 3068))

        def body(_, w):
            m = jnp.max(w, axis=0, keepdims=True)
            idx = jnp.min(jnp.where(w == m, fidx, 2 * _NF), axis=0,
                          keepdims=True)
            return jnp.where(fidx == idx, -1.0, w)

        work = jax.lax.fori_loop(0, _K, body, amp)
        sel = work < 0.0
        spec = jnp.concatenate([
            jnp.where(jnp.concatenate([sel[:512], sel[:512]], axis=0), odd, 0.0),
            jnp.where(jnp.concatenate([sel[512:768], sel[512:768]], axis=0), e2, 0.0),
            jnp.where(jnp.concatenate([sel[768:], sel[768:]], axis=0), e4, 0.0),
        ], axis=0)
        spec_ref[0] = spec.astype(jnp.bfloat16)


def _bdot(a, b):
    return jax.lax.dot_general(
        a, b, (((1,), (0,)), ((), ())),
        preferred_element_type=jnp.float32)


def _recon_kernel(tob_ref, te2b_ref, te4b_ref, spec_ref, o_ref):
    spec = spec_ref[0]
    so = _bdot(tob_ref[...], spec[:1024])
    se2 = _bdot(te2b_ref[...], spec[1024:1536])
    se4 = _bdot(te4b_ref[...], spec[1536:])
    se = jnp.concatenate([se4 + se2, se4 - se2], axis=0)
    o_ref[0] = jnp.concatenate([se + so, se - so], axis=0)


def kernel(Z):
    Bs, Ls, Ds = Z.shape
    to = jnp.asarray(_T_ODD)
    te2 = jnp.asarray(_T_E2)
    te4 = jnp.asarray(_T_E4)
    tob = jnp.asarray(_T_ODD.reshape(1024, 1024).astype(jnp.bfloat16))
    te2b = jnp.asarray(_T_E2.astype(jnp.bfloat16))
    te4b = jnp.asarray(_T_E4.astype(jnp.bfloat16))
    zp = Z.reshape(Bs, 4, Ls // 4, Ds)
    tch = Ls // 4 // _NTC

    spec = pl.pallas_call(
        _dft_select_kernel,
        grid=(Bs, _NTC),
        in_specs=[
            pl.BlockSpec((2, tch, 1024), lambda b, tc: (0, tc, 0)),
            pl.BlockSpec((tch, 512), lambda b, tc: (tc, 0)),
            pl.BlockSpec((tch, 512), lambda b, tc: (tc, 0)),
            pl.BlockSpec((1, 4, tch, Ds), lambda b, tc: (b, 0, tc, 0)),
        ],
        out_specs=pl.BlockSpec((1, 2 * _NF, Ds), lambda b, tc: (b, 0, 0)),
        out_shape=jax.ShapeDtypeStruct((Bs, 2 * _NF, Ds), jnp.bfloat16),
        scratch_shapes=[pltpu.VMEM((_NF, Ds), jnp.float32),
                        pltpu.VMEM((512, Ds), jnp.float32),
                        pltpu.VMEM((512, Ds), jnp.float32)],
        compiler_params=pltpu.CompilerParams(
            dimension_semantics=("parallel", "arbitrary")),
    )(to, te2, te4, zp)

    return pl.pallas_call(
        _recon_kernel,
        grid=(Bs,),
        in_specs=[
            pl.BlockSpec((1024, 1024), lambda b: (0, 0)),
            pl.BlockSpec((512, 512), lambda b: (0, 0)),
            pl.BlockSpec((512, 512), lambda b: (0, 0)),
            pl.BlockSpec((1, 2 * _NF, Ds), lambda b: (b, 0, 0)),
        ],
        out_specs=pl.BlockSpec((1, _L, Ds), lambda b: (b, 0, 0)),
        out_shape=jax.ShapeDtypeStruct((Bs, Ls, Ds), jnp.float32),
        compiler_params=pltpu.CompilerParams(
            dimension_semantics=("arbitrary",)),
    )(tob, te2b, te4b, spec)


# speedup vs baseline: 11.7725x; 1.1017x over previous
import numpy as np

import jax
import jax.numpy as jnp
from jax.experimental import pallas as pl
from jax.experimental.pallas import tpu as pltpu

_L = 2048
_NF = 1024
_K = 5


def _build_tables():
    n1 = np.arange(1024, dtype=np.float64)[:, None]
    j1 = np.arange(512, dtype=np.float64)[None, :]
    ang_o = (2.0 * np.pi / 2048.0) * (2.0 * j1 + 1.0) * n1
    t_odd = np.concatenate([np.cos(ang_o), np.sin(ang_o)], axis=1)

    n2 = np.arange(512, dtype=np.float64)[:, None]
    j2 = np.arange(256, dtype=np.float64)[None, :]
    ang_e2 = (2.0 * np.pi / 1024.0) * (2.0 * j2 + 1.0) * n2
    t_e2 = np.concatenate([np.cos(ang_e2), np.sin(ang_e2)], axis=1)
    ang_e4 = (2.0 * np.pi / 512.0) * (j2 + 1.0) * n2
    t_e4 = np.concatenate([np.cos(ang_e4), np.sin(ang_e4)], axis=1)

    return (t_odd.astype(np.float32).reshape(2, 512, 1024),
            t_e2.astype(np.float32),
            t_e4.astype(np.float32))


_T_ODD, _T_E2, _T_E4 = _build_tables()


def _hdot(a, b):
    return jax.lax.dot_general(
        a, b, (((0,), (0,)), ((), ())),
        preferred_element_type=jnp.float32,
        precision=jax.lax.Precision.HIGHEST)


def _bdot(a, b):
    return jax.lax.dot_general(
        a, b, (((1,), (0,)), ((), ())),
        preferred_element_type=jnp.float32)


def _freq_attn_kernel(to_ref, te2_ref, te4_ref, tob_ref, te2b_ref, te4b_ref,
                      z_ref, o_ref):
    zb = z_ref[0]
    q0, q1, q2, q3 = zb[0], zb[1], zb[2], zb[3]
    c0 = q0 - q2
    c1 = q1 - q3
    a0 = q0 + q2
    a1 = q1 + q3
    ap = a0 + a1
    cp = a0 - a1

    odd = _hdot(to_ref[0], c0) + _hdot(to_ref[1], c1)
    e2 = _hdot(te2_ref[...], cp)
    e4 = _hdot(te4_ref[...], ap)

    Re = jnp.concatenate([odd[:512], e2[:256], e4[:256]], axis=0)
    Im = jnp.concatenate([odd[512:], e2[256:], e4[256:]], axis=0)
    amp = jnp.sqrt(Re * Re + Im * Im)
    r = jax.lax.broadcasted_iota(jnp.int32, amp.shape, 0)
    fidx = jnp.where(r < 512, 2 * r + 1,
                     jnp.where(r < 768, 4 * r - 2046, 4 * r - 3068))

    def body(_, w):
        m = jnp.max(w, axis=0, keepdims=True)
        idx = jnp.min(jnp.where(w == m, fidx, 2 * _NF), axis=0, keepdims=True)
        return jnp.where(fidx == idx, -1.0, w)

    work = jax.lax.fori_loop(0, _K, body, amp)
    sel = work < 0.0

    spec_o = jnp.where(jnp.concatenate([sel[:512], sel[:512]], axis=0),
                       odd, 0.0).astype(jnp.bfloat16)
    spec_e2 = jnp.where(jnp.concatenate([sel[512:768], sel[512:768]], axis=0),
                        e2, 0.0).astype(jnp.bfloat16)
    spec_e4 = jnp.where(jnp.concatenate([sel[768:], sel[768:]], axis=0),
                        e4, 0.0).astype(jnp.bfloat16)

    so = _bdot(tob_ref[...], spec_o)
    se2 = _bdot(te2b_ref[...], spec_e2)
    se4 = _bdot(te4b_ref[...], spec_e4)
    se = jnp.concatenate([se4 + se2, se4 - se2], axis=0)
    o_ref[0] = jnp.concatenate([se + so, se - so], axis=0)


def kernel(Z):
    Bs, Ls, Ds = Z.shape
    to = jnp.asarray(_T_ODD)
    te2 = jnp.asarray(_T_E2)
    te4 = jnp.asarray(_T_E4)
    tob = jnp.asarray(_T_ODD.reshape(1024, 1024).astype(jnp.bfloat16))
    te2b = jnp.asarray(_T_E2.astype(jnp.bfloat16))
    te4b = jnp.asarray(_T_E4.astype(jnp.bfloat16))
    zp = Z.reshape(Bs, 4, Ls // 4, Ds)

    return pl.pallas_call(
        _freq_attn_kernel,
        grid=(Bs,),
        in_specs=[
            pl.BlockSpec((2, 512, 1024), lambda b: (0, 0, 0)),
            pl.BlockSpec((512, 512), lambda b: (0, 0)),
            pl.BlockSpec((512, 512), lambda b: (0, 0)),
            pl.BlockSpec((1024, 1024), lambda b: (0, 0)),
            pl.BlockSpec((512, 512), lambda b: (0, 0)),
            pl.BlockSpec((512, 512), lambda b: (0, 0)),
            pl.BlockSpec((1, 4, Ls // 4, Ds), lambda b: (b, 0, 0, 0)),
        ],
        out_specs=pl.BlockSpec((1, Ls, Ds), lambda b: (b, 0, 0)),
        out_shape=jax.ShapeDtypeStruct((Bs, Ls, Ds), jnp.float32),
        compiler_params=pltpu.CompilerParams(
            dimension_semantics=("arbitrary",)),
    )(to, te2, te4, tob, te2b, te4b, zp)


# speedup vs baseline: 11.8159x; 1.0037x over previous
import numpy as np

import jax
import jax.numpy as jnp
from jax.experimental import pallas as pl
from jax.experimental.pallas import tpu as pltpu

_L = 2048
_NF = 1024
_K = 5


def _build_tables():
    n1 = np.arange(1024, dtype=np.float64)[:, None]
    j1 = np.arange(512, dtype=np.float64)[None, :]
    ang_o = (2.0 * np.pi / 2048.0) * (2.0 * j1 + 1.0) * n1
    t_odd = np.concatenate([np.cos(ang_o), np.sin(ang_o)], axis=1)

    n2 = np.arange(512, dtype=np.float64)[:, None]
    j2 = np.arange(256, dtype=np.float64)[None, :]
    ang_e2 = (2.0 * np.pi / 1024.0) * (2.0 * j2 + 1.0) * n2
    t_e2 = np.concatenate([np.cos(ang_e2), np.sin(ang_e2)], axis=1)
    ang_e4 = (2.0 * np.pi / 512.0) * (j2 + 1.0) * n2
    t_e4 = np.concatenate([np.cos(ang_e4), np.sin(ang_e4)], axis=1)

    return (t_odd.astype(np.float32).reshape(2, 512, 1024),
            t_e2.astype(np.float32),
            t_e4.astype(np.float32))


_T_ODD, _T_E2, _T_E4 = _build_tables()


def _hdot(a, b):
    return jax.lax.dot_general(
        a, b, (((0,), (0,)), ((), ())),
        preferred_element_type=jnp.float32,
        precision=jax.lax.Precision.HIGHEST)


def _bdot(a, b):
    return jax.lax.dot_general(
        a, b, (((1,), (0,)), ((), ())),
        preferred_element_type=jnp.float32)


def _freq_attn_kernel(to_ref, te2_ref, te4_ref, tob_ref, te2b_ref, te4b_ref,
                      z_ref, o_ref):
    zb = z_ref[0]
    q0, q1, q2, q3 = zb[0], zb[1], zb[2], zb[3]
    c0 = q0 - q2
    c1 = q1 - q3
    a0 = q0 + q2
    a1 = q1 + q3
    ap = a0 + a1
    cp = a0 - a1

    odd = _hdot(to_ref[0], c0) + _hdot(to_ref[1], c1)
    e2 = _hdot(te2_ref[...], cp)
    e4 = _hdot(te4_ref[...], ap)

    Re = jnp.concatenate([odd[:512], e2[:256], e4[:256]], axis=0)
    Im = jnp.concatenate([odd[512:], e2[256:], e4[256:]], axis=0)
    amp = jnp.sqrt(Re * Re + Im * Im)
    r = jax.lax.broadcasted_iota(jnp.int32, amp.shape, 0)
    fidx = jnp.where(r < 512, 2 * r + 1,
                     jnp.where(r < 768, 4 * r - 2046, 4 * r - 3068))

    def body(_, w):
        m = jnp.max(w, axis=0, keepdims=True)
        idx = jnp.min(jnp.where(w == m, fidx, 2 * _NF), axis=0, keepdims=True)
        return jnp.where(fidx == idx, -1.0, w)

    work = jax.lax.fori_loop(0, _K, body, amp)
    sel = work < 0.0

    spec_o = jnp.where(jnp.concatenate([sel[:512], sel[:512]], axis=0),
                       odd, 0.0).astype(jnp.bfloat16)
    spec_e2 = jnp.where(jnp.concatenate([sel[512:768], sel[512:768]], axis=0),
                        e2, 0.0).astype(jnp.bfloat16)
    spec_e4 = jnp.where(jnp.concatenate([sel[768:], sel[768:]], axis=0),
                        e4, 0.0).astype(jnp.bfloat16)

    so = _bdot(tob_ref[...], spec_o)
    se2 = _bdot(te2b_ref[...], spec_e2)
    se4 = _bdot(te4b_ref[...], spec_e4)
    se = jnp.concatenate([se4 + se2, se4 - se2], axis=0)
    o_ref[0] = jnp.concatenate([se + so, se - so], axis=0)


def kernel(Z):
    Bs, Ls, Ds = Z.shape
    to = jnp.asarray(_T_ODD)
    te2 = jnp.asarray(_T_E2)
    te4 = jnp.asarray(_T_E4)
    tob = jnp.asarray(_T_ODD.reshape(1024, 1024).astype(jnp.bfloat16))
    te2b = jnp.asarray(_T_E2.astype(jnp.bfloat16))
    te4b = jnp.asarray(_T_E4.astype(jnp.bfloat16))
    zp = Z.reshape(Bs, 4, Ls // 4, Ds)

    dt = 512
    return pl.pallas_call(
        _freq_attn_kernel,
        grid=(Bs, Ds // dt),
        in_specs=[
            pl.BlockSpec((2, 512, 1024), lambda b, j: (0, 0, 0)),
            pl.BlockSpec((512, 512), lambda b, j: (0, 0)),
            pl.BlockSpec((512, 512), lambda b, j: (0, 0)),
            pl.BlockSpec((1024, 1024), lambda b, j: (0, 0)),
            pl.BlockSpec((512, 512), lambda b, j: (0, 0)),
            pl.BlockSpec((512, 512), lambda b, j: (0, 0)),
            pl.BlockSpec((1, 4, Ls // 4, dt), lambda b, j: (b, 0, 0, j)),
        ],
        out_specs=pl.BlockSpec((1, Ls, dt), lambda b, j: (b, 0, j)),
        out_shape=jax.ShapeDtypeStruct((Bs, Ls, Ds), jnp.float32),
        compiler_params=pltpu.CompilerParams(
            dimension_semantics=("parallel", "parallel")),
    )(to, te2, te4, tob, te2b, te4b, zp)
